# Initial kernel scaffold; baseline (speedup 1.0000x reference)
import jax
import jax.numpy as jnp
from jax.experimental import pallas as pl


def kernel(edge_index, t, msg, memory, last_update, w_t, b_t, W_nbr, W_self, W_upd):
    raise NotImplementedError("write your pallas kernel here")


# speedup vs baseline: 2.5080x; 2.5080x over previous
import functools

import jax
import jax.numpy as jnp
from jax.experimental import pallas as pl

N = 100000
B = 32768
D_MEM = 128
D_MSG = 16
D_TIME = 32
D_OUT = 128

_BLK = 2048


def _dense_body(mem_src_ref, mem_dst_ref, msg_ref, t_ref, lu_src_ref,
                w_t_ref, b_t_ref, W_nbr_ref, W_self_ref, W_upd_ref,
                m_ref, upd_ref, hs_ref, hd_ref):
    mem_src = mem_src_ref[...]
    mem_dst = mem_dst_ref[...]
    msg = msg_ref[...]
    rel = lu_src_ref[...] - t_ref[...]
    w_t = w_t_ref[...]
    b_t = b_t_ref[...]
    enc_rel = jnp.cos(rel * w_t + b_t)
    enc_t = jnp.cos((-rel) * w_t + b_t)
    nbr_in = jnp.concatenate([mem_src, msg, enc_rel], axis=1)
    m_ref[...] = jax.nn.relu(
        jax.lax.dot(nbr_in, W_nbr_ref[...],
                    preferred_element_type=jnp.float32))
    upd_in = jnp.concatenate([mem_src, mem_dst, msg, enc_t], axis=1)
    upd_ref[...] = jnp.tanh(
        jax.lax.dot(upd_in, W_upd_ref[...],
                    preferred_element_type=jnp.float32))
    hs_ref[...] = jax.lax.dot(mem_src, W_self_ref[...],
                              preferred_element_type=jnp.float32)
    hd_ref[...] = jax.lax.dot(mem_dst, W_self_ref[...],
                              preferred_element_type=jnp.float32)


def _dense(mem_src, mem_dst, msg, t, lu_src, w_t, b_t, W_nbr, W_self, W_upd):
    grid = (B // _BLK,)
    row_spec = lambda d: pl.BlockSpec((_BLK, d), lambda i: (i, 0))
    full = lambda a, b: pl.BlockSpec((a, b), lambda i: (0, 0))
    return pl.pallas_call(
        _dense_body,
        grid=grid,
        in_specs=[
            row_spec(D_MEM), row_spec(D_MEM), row_spec(D_MSG),
            row_spec(1), row_spec(1),
            full(1, D_TIME), full(1, D_TIME),
            full(D_MEM + D_MSG + D_TIME, D_OUT),
            full(D_MEM, D_OUT),
            full(2 * D_MEM + D_MSG + D_TIME, D_MEM),
        ],
        out_specs=[row_spec(D_OUT), row_spec(D_MEM),
                   row_spec(D_OUT), row_spec(D_OUT)],
        out_shape=[
            jax.ShapeDtypeStruct((B, D_OUT), jnp.float32),
            jax.ShapeDtypeStruct((B, D_MEM), jnp.float32),
            jax.ShapeDtypeStruct((B, D_OUT), jnp.float32),
            jax.ShapeDtypeStruct((B, D_OUT), jnp.float32),
        ],
    )(mem_src, mem_dst, msg, t, lu_src, w_t, b_t, W_nbr, W_self, W_upd)


def kernel(edge_index, t, msg, memory, last_update, w_t, b_t, W_nbr, W_self, W_upd):
    src, dst = edge_index[0], edge_index[1]
    mem_src = memory[src]
    mem_dst = memory[dst]
    lu_src = last_update[src]
    t2 = t[:, None]
    lu2 = lu_src[:, None]
    m, upd, hs_lin, hd_lin = _dense(
        mem_src, mem_dst, msg, t2, lu2, w_t, b_t.reshape(1, D_TIME),
        W_nbr, W_self, W_upd)
    agg = jnp.zeros((N, D_OUT), jnp.float32).at[dst].add(m)
    h_src = jax.nn.relu(hs_lin + agg[src])
    h_dst = jax.nn.relu(hd_lin + agg[dst])
    new_memory = memory.at[src].set(upd)
    new_last_update = last_update.at[src].set(t)
    return (h_src, h_dst, new_memory, new_last_update)


# speedup vs baseline: 2.5247x; 1.0066x over previous
import functools

import jax
import jax.numpy as jnp
from jax import lax
from jax.experimental import pallas as pl
from jax.experimental.pallas import tpu as pltpu
from jax.experimental.pallas import tpu_sc as plsc

N = 100000
B = 32768
D_MEM = 128
D_MSG = 16
D_TIME = 32
D_OUT = 128

_BLK = 2048


def _dense_body(mem_src_ref, mem_dst_ref, msg_ref, t_ref, lu_src_ref,
                w_t_ref, b_t_ref, W_nbr_ref, W_self_ref, W_upd_ref,
                m_ref, upd_ref, hs_ref, hd_ref):
    mem_src = mem_src_ref[...]
    mem_dst = mem_dst_ref[...]
    msg = msg_ref[...]
    rel = lu_src_ref[...] - t_ref[...]
    w_t = w_t_ref[...]
    b_t = b_t_ref[...]
    enc_rel = jnp.cos(rel * w_t + b_t)
    enc_t = jnp.cos((-rel) * w_t + b_t)
    nbr_in = jnp.concatenate([mem_src, msg, enc_rel], axis=1)
    m_ref[...] = jax.nn.relu(
        jax.lax.dot(nbr_in, W_nbr_ref[...],
                    preferred_element_type=jnp.float32))
    upd_in = jnp.concatenate([mem_src, mem_dst, msg, enc_t], axis=1)
    upd_ref[...] = jnp.tanh(
        jax.lax.dot(upd_in, W_upd_ref[...],
                    preferred_element_type=jnp.float32))
    hs_ref[...] = jax.lax.dot(mem_src, W_self_ref[...],
                              preferred_element_type=jnp.float32)
    hd_ref[...] = jax.lax.dot(mem_dst, W_self_ref[...],
                              preferred_element_type=jnp.float32)


def _dense(mem_src, mem_dst, msg, t, lu_src, w_t, b_t, W_nbr, W_self, W_upd):
    grid = (B // _BLK,)
    row_spec = lambda d: pl.BlockSpec((_BLK, d), lambda i: (i, 0))
    full = lambda a, b: pl.BlockSpec((a, b), lambda i: (0, 0))
    return pl.pallas_call(
        _dense_body,
        grid=grid,
        in_specs=[
            row_spec(D_MEM), row_spec(D_MEM), row_spec(D_MSG),
            row_spec(1), row_spec(1),
            full(1, D_TIME), full(1, D_TIME),
            full(D_MEM + D_MSG + D_TIME, D_OUT),
            full(D_MEM, D_OUT),
            full(2 * D_MEM + D_MSG + D_TIME, D_MEM),
        ],
        out_specs=[row_spec(D_OUT), row_spec(D_MEM),
                   row_spec(D_OUT), row_spec(D_OUT)],
        out_shape=[
            jax.ShapeDtypeStruct((B, D_OUT), jnp.float32),
            jax.ShapeDtypeStruct((B, D_MEM), jnp.float32),
            jax.ShapeDtypeStruct((B, D_OUT), jnp.float32),
            jax.ShapeDtypeStruct((B, D_OUT), jnp.float32),
        ],
    )(mem_src, mem_dst, msg, t, lu_src, w_t, b_t, W_nbr, W_self, W_upd)


_INFO = plsc.get_sparse_core_info()
_NC, _NS = _INFO.num_cores, _INFO.num_subcores
_NW = _NC * _NS
_EV_W = B // _NW
_CH = 512


def _gather_body(mem_hbm, ei_hbm, lu_hbm, ms_out, md_out, lus_out,
                 idx_v, rows_v, lu_v, sem):
    wid = lax.axis_index("s") * _NC + lax.axis_index("c")
    base = wid * _EV_W
    pltpu.sync_copy(ei_hbm.at[0, pl.ds(base, _EV_W)], idx_v)
    pltpu.async_copy(lu_hbm.at[idx_v], lu_v, sem).wait()
    pltpu.sync_copy(lu_v, lus_out.at[pl.ds(base, _EV_W)])
    for half in range(_EV_W // _CH):
        off = half * _CH
        pltpu.async_copy(mem_hbm.at[idx_v.at[pl.ds(off, _CH)]], rows_v,
                         sem).wait()
        pltpu.sync_copy(rows_v, ms_out.at[pl.ds(base + off, _CH)])
    pltpu.sync_copy(ei_hbm.at[1, pl.ds(base, _EV_W)], idx_v)
    for half in range(_EV_W // _CH):
        off = half * _CH
        pltpu.async_copy(mem_hbm.at[idx_v.at[pl.ds(off, _CH)]], rows_v,
                         sem).wait()
        pltpu.sync_copy(rows_v, md_out.at[pl.ds(base + off, _CH)])


_sc_gather = pl.kernel(
    _gather_body,
    out_type=[
        jax.ShapeDtypeStruct((B, D_MEM), jnp.float32),
        jax.ShapeDtypeStruct((B, D_MEM), jnp.float32),
        jax.ShapeDtypeStruct((B,), jnp.float32),
    ],
    mesh=plsc.VectorSubcoreMesh(core_axis_name="c", subcore_axis_name="s"),
    scratch_types=[
        pltpu.VMEM((_EV_W,), jnp.int32),
        pltpu.VMEM((_CH, D_MEM), jnp.float32),
        pltpu.VMEM((_EV_W,), jnp.float32),
        pltpu.SemaphoreType.DMA,
    ],
)


def kernel(edge_index, t, msg, memory, last_update, w_t, b_t, W_nbr, W_self, W_upd):
    src, dst = edge_index[0], edge_index[1]
    mem_src, mem_dst, lu_src = _sc_gather(memory, edge_index, last_update)
    t2 = t[:, None]
    lu2 = lu_src[:, None]
    m, upd, hs_lin, hd_lin = _dense(
        mem_src, mem_dst, msg, t2, lu2, w_t, b_t.reshape(1, D_TIME),
        W_nbr, W_self, W_upd)
    agg = jnp.zeros((N, D_OUT), jnp.float32).at[dst].add(m)
    h_src = jax.nn.relu(hs_lin + agg[src])
    h_dst = jax.nn.relu(hd_lin + agg[dst])
    new_memory = memory.at[src].set(upd)
    new_last_update = last_update.at[src].set(t)
    return (h_src, h_dst, new_memory, new_last_update)


# speedup vs baseline: 2.6707x; 1.0578x over previous
import functools

import jax
import jax.numpy as jnp
from jax import lax
from jax.experimental import pallas as pl
from jax.experimental.pallas import tpu as pltpu
from jax.experimental.pallas import tpu_sc as plsc

N = 100000
B = 32768
D_MEM = 128
D_MSG = 16
D_TIME = 32
D_OUT = 128

_BLK = 2048


def _dense_body(mem_src_ref, mem_dst_ref, msg_ref, t_ref, lu_src_ref,
                w_t_ref, b_t_ref, W_nbr_ref, W_self_ref, W_upd_ref,
                m_ref, upd_ref, hs_ref, hd_ref):
    mem_src = mem_src_ref[...]
    mem_dst = mem_dst_ref[...]
    msg = msg_ref[...]
    rel = lu_src_ref[...] - t_ref[...]
    w_t = w_t_ref[...]
    b_t = b_t_ref[...]
    enc_rel = jnp.cos(rel * w_t + b_t)
    enc_t = jnp.cos((-rel) * w_t + b_t)
    nbr_in = jnp.concatenate([mem_src, msg, enc_rel], axis=1)
    m_ref[...] = jax.nn.relu(
        jax.lax.dot(nbr_in, W_nbr_ref[...],
                    preferred_element_type=jnp.float32))
    upd_in = jnp.concatenate([mem_src, mem_dst, msg, enc_t], axis=1)
    upd_ref[...] = jnp.tanh(
        jax.lax.dot(upd_in, W_upd_ref[...],
                    preferred_element_type=jnp.float32))
    hs_ref[...] = jax.lax.dot(mem_src, W_self_ref[...],
                              preferred_element_type=jnp.float32)
    hd_ref[...] = jax.lax.dot(mem_dst, W_self_ref[...],
                              preferred_element_type=jnp.float32)


def _dense(mem_src, mem_dst, msg, t, lu_src, w_t, b_t, W_nbr, W_self, W_upd):
    grid = (B // _BLK,)
    row_spec = lambda d: pl.BlockSpec((_BLK, d), lambda i: (i, 0))
    full = lambda a, b: pl.BlockSpec((a, b), lambda i: (0, 0))
    return pl.pallas_call(
        _dense_body,
        grid=grid,
        in_specs=[
            row_spec(D_MEM), row_spec(D_MEM), row_spec(D_MSG),
            row_spec(1), row_spec(1),
            full(1, D_TIME), full(1, D_TIME),
            full(D_MEM + D_MSG + D_TIME, D_OUT),
            full(D_MEM, D_OUT),
            full(2 * D_MEM + D_MSG + D_TIME, D_MEM),
        ],
        out_specs=[row_spec(D_OUT), row_spec(D_MEM),
                   row_spec(D_OUT), row_spec(D_OUT)],
        out_shape=[
            jax.ShapeDtypeStruct((B, D_OUT), jnp.float32),
            jax.ShapeDtypeStruct((B, D_MEM), jnp.float32),
            jax.ShapeDtypeStruct((B, D_OUT), jnp.float32),
            jax.ShapeDtypeStruct((B, D_OUT), jnp.float32),
        ],
    )(mem_src, mem_dst, msg, t, lu_src, w_t, b_t, W_nbr, W_self, W_upd)


_INFO = plsc.get_sparse_core_info()
_NC, _NS = _INFO.num_cores, _INFO.num_subcores
_NW = _NC * _NS
_EV_W = B // _NW
_CH = 512


def _gather_body(mem_hbm, ei_hbm, lu_hbm, ms_out, md_out, lus_out,
                 idx_v, rows_v, lu_v, sem):
    wid = lax.axis_index("s") * _NC + lax.axis_index("c")
    base = wid * _EV_W
    pltpu.sync_copy(ei_hbm.at[0, pl.ds(base, _EV_W)], idx_v)
    pltpu.async_copy(lu_hbm.at[idx_v], lu_v, sem).wait()
    pltpu.sync_copy(lu_v, lus_out.at[pl.ds(base, _EV_W)])
    for half in range(_EV_W // _CH):
        off = half * _CH
        pltpu.async_copy(mem_hbm.at[idx_v.at[pl.ds(off, _CH)]], rows_v,
                         sem).wait()
        pltpu.sync_copy(rows_v, ms_out.at[pl.ds(base + off, _CH)])
    pltpu.sync_copy(ei_hbm.at[1, pl.ds(base, _EV_W)], idx_v)
    for half in range(_EV_W // _CH):
        off = half * _CH
        pltpu.async_copy(mem_hbm.at[idx_v.at[pl.ds(off, _CH)]], rows_v,
                         sem).wait()
        pltpu.sync_copy(rows_v, md_out.at[pl.ds(base + off, _CH)])


_sc_gather = pl.kernel(
    _gather_body,
    out_type=[
        jax.ShapeDtypeStruct((B, D_MEM), jnp.float32),
        jax.ShapeDtypeStruct((B, D_MEM), jnp.float32),
        jax.ShapeDtypeStruct((B,), jnp.float32),
    ],
    mesh=plsc.VectorSubcoreMesh(core_axis_name="c", subcore_axis_name="s"),
    scratch_types=[
        pltpu.VMEM((_EV_W,), jnp.int32),
        pltpu.VMEM((_CH, D_MEM), jnp.float32),
        pltpu.VMEM((_EV_W,), jnp.float32),
        pltpu.SemaphoreType.DMA,
    ],
)


_CHK = 128
_NFULL = N // _CHK
_REM = N - _NFULL * _CHK
_REM_OWNER = _NFULL % _NW
_KMAX = _NFULL // _NW + 1
_PCAP = _KMAX * _CHK
_STRIP = 2048
_NSTRIP = B // _STRIP


def _update_body(mem_hbm, ei_hbm, t_hbm, lu_hbm, upd_hbm,
                 newmem_hbm, newlu_hbm,
                 ss, cv, cb, pos, wn1, wp1, wn2, rows, wrows, luv, twv, sem):
    wid = lax.axis_index("s") * _NC + lax.axis_index("c")
    iota = lax.iota(jnp.int32, 16)
    neg1 = jnp.zeros((16,), jnp.int32) - 1

    def p1(i, _):
        pos[pl.ds(i * 16, 16)] = neg1
        return 0
    lax.fori_loop(0, _PCAP // 16, p1, 0)

    def p2(s, _):
        pltpu.sync_copy(ei_hbm.at[0, pl.ds(s * _STRIP, _STRIP)], ss)

        def compact(j, off):
            v = ss[pl.ds(j * 16, 16)]
            own = ((v >> 7) & (_NW - 1)) == wid
            slot = ((v >> 12) << 7) | (v & (_CHK - 1))
            bv = s * _STRIP + j * 16 + iota
            plsc.store_compressed(cv.at[pl.ds(off, 16)], slot, mask=own)
            plsc.store_compressed(cb.at[pl.ds(off, 16)], bv, mask=own)
            return off + jnp.sum(own.astype(jnp.int32))
        off = lax.fori_loop(0, _STRIP // 16, compact, jnp.int32(0))

        def scat(k, _):
            sl = cv[pl.ds(k * 16, 16)]
            bvv = cb[pl.ds(k * 16, 16)]
            valid = (k * 16 + iota) < off
            _, lastm = plsc.scan_count(sl, valid)
            plsc.store_scatter(pos, [sl], bvv, mask=lastm & valid)
            return 0
        lax.fori_loop(0, (off + 15) // 16, scat, 0)
        return 0
    lax.fori_loop(0, _NSTRIP, p2, 0)

    def p3(j, w):
        p = pos[pl.ds(j * 16, 16)]
        m = p >= 0
        sidx = j * 16 + iota
        node = wid * _CHK + (sidx >> 7) * (_NW * _CHK) + (sidx & (_CHK - 1))
        plsc.store_compressed(wn1.at[pl.ds(w, 16)], node, mask=m)
        plsc.store_compressed(wp1.at[pl.ds(w, 16)], p, mask=m)
        return w + jnp.sum(m.astype(jnp.int32))
    w_cnt = lax.fori_loop(0, _PCAP // 16, p3, jnp.int32(0))

    n_wchunk = (w_cnt + _CHK - 1) // _CHK

    @pl.when(w_cnt > 0)
    def _():
        z16 = jnp.zeros((16,), jnp.int32)
        padn = plsc.load_gather(wn1, [z16])
        padp = plsc.load_gather(wp1, [z16])

        def pad(j, _):
            mval = (j * 16 + iota) < w_cnt
            wn1[pl.ds(j * 16, 16)] = jnp.where(mval, wn1[pl.ds(j * 16, 16)],
                                               padn)
            wp1[pl.ds(j * 16, 16)] = jnp.where(mval, wp1[pl.ds(j * 16, 16)],
                                               padp)
            return 0
        lax.fori_loop(w_cnt // 16, n_wchunk * (_CHK // 16), pad, 0)

        def tocol(j, _):
            wn2[j // 8, pl.ds((j % 8) * 16, 16)] = wn1[pl.ds(j * 16, 16)]
            return 0
        lax.fori_loop(0, n_wchunk * (_CHK // 16), tocol, 0)

    nk = (_NFULL + _NW - 1 - wid) // _NW

    def copy_chunk(k, _):
        r0 = (wid + k * _NW) * _CHK
        pltpu.sync_copy(mem_hbm.at[pl.ds(r0, _CHK)], rows)
        pltpu.sync_copy(rows, newmem_hbm.at[pl.ds(r0, _CHK)])
        pltpu.sync_copy(lu_hbm.at[pl.ds(r0, _CHK)], luv)
        pltpu.sync_copy(luv, newlu_hbm.at[pl.ds(r0, _CHK)])
        return 0
    lax.fori_loop(0, nk, copy_chunk, 0)

    @pl.when(wid == _REM_OWNER)
    def _():
        r0 = _NFULL * _CHK
        pltpu.sync_copy(mem_hbm.at[pl.ds(r0, _REM)], rows.at[pl.ds(0, _REM)])
        pltpu.sync_copy(rows.at[pl.ds(0, _REM)], newmem_hbm.at[pl.ds(r0, _REM)])
        pltpu.sync_copy(lu_hbm.at[pl.ds(r0, _REM)], luv.at[pl.ds(0, _REM)])
        pltpu.sync_copy(luv.at[pl.ds(0, _REM)], newlu_hbm.at[pl.ds(r0, _REM)])

    def winner_chunk(c2, _):
        pltpu.async_copy(upd_hbm.at[wp1.at[pl.ds(c2 * _CHK, _CHK)]], wrows,
                         sem).wait()
        pltpu.sync_copy(wrows, newmem_hbm.at[wn2.at[c2]])
        pltpu.async_copy(t_hbm.at[wp1.at[pl.ds(c2 * _CHK, _CHK)]], twv,
                         sem).wait()
        pltpu.sync_copy(twv, newlu_hbm.at[wn2.at[c2]])
        return 0
    lax.fori_loop(0, n_wchunk, winner_chunk, 0)


_sc_update = pl.kernel(
    _update_body,
    out_type=[
        jax.ShapeDtypeStruct((N, D_MEM), jnp.float32),
        jax.ShapeDtypeStruct((N,), jnp.float32),
    ],
    mesh=plsc.VectorSubcoreMesh(core_axis_name="c", subcore_axis_name="s"),
    scratch_types=[
        pltpu.VMEM((_STRIP,), jnp.int32),
        pltpu.VMEM((_STRIP + 16,), jnp.int32),
        pltpu.VMEM((_STRIP + 16,), jnp.int32),
        pltpu.VMEM((_PCAP,), jnp.int32),
        pltpu.VMEM((_PCAP + 16,), jnp.int32),
        pltpu.VMEM((_PCAP + 16,), jnp.int32),
        pltpu.VMEM((_KMAX, _CHK), jnp.int32),
        pltpu.VMEM((_CHK, D_MEM), jnp.float32),
        pltpu.VMEM((_CHK, D_MEM), jnp.float32),
        pltpu.VMEM((_CHK,), jnp.float32),
        pltpu.VMEM((_CHK,), jnp.float32),
        pltpu.SemaphoreType.DMA,
    ],
    compiler_params=pltpu.CompilerParams(needs_layout_passes=False),
)


def kernel(edge_index, t, msg, memory, last_update, w_t, b_t, W_nbr, W_self, W_upd):
    src, dst = edge_index[0], edge_index[1]
    mem_src, mem_dst, lu_src = _sc_gather(memory, edge_index, last_update)
    t2 = t[:, None]
    lu2 = lu_src[:, None]
    m, upd, hs_lin, hd_lin = _dense(
        mem_src, mem_dst, msg, t2, lu2, w_t, b_t.reshape(1, D_TIME),
        W_nbr, W_self, W_upd)
    agg = jnp.zeros((N, D_OUT), jnp.float32).at[dst].add(m)
    h_src = jax.nn.relu(hs_lin + agg[src])
    h_dst = jax.nn.relu(hd_lin + agg[dst])
    new_memory, new_last_update = _sc_update(
        memory, edge_index, t, last_update, upd)
    return (h_src, h_dst, new_memory, new_last_update)
